# baseline (speedup 1.0000x reference)
import functools

import jax
import jax.numpy as jnp
from jax import lax
from jax.experimental import pallas as pl
from jax.experimental.pallas import tpu as pltpu
from jax.experimental.pallas import tpu_sc as plsc

_N = 10000
_E = 320000
_D = 128

_NC = 2
_NS = 16
_EP_TILE = _E // (_NC * _NS)
_CHUNK = 64
_NCHUNK = 162
_EPAD = _NCHUNK * _CHUNK
_NPAD = 10240
_ROWS_PT = _NPAD // _NS
_DUMP = _NPAD - 1

_NBUF = 3
_NIB = 2 * _NBUF


def _lift_body(x_ref, w_ref, b_ref, o_ref):
    o_ref[...] = (
        jnp.dot(x_ref[...], w_ref[...], preferred_element_type=jnp.float32)
        + b_ref[...]
    )


def _lift(x, w, b):
    blk = 1000
    return pl.pallas_call(
        _lift_body,
        grid=(_N // blk,),
        in_specs=[
            pl.BlockSpec((blk, _D), lambda i: (i, 0)),
            pl.BlockSpec((_D, _D), lambda i: (0, 0)),
            pl.BlockSpec((1, _D), lambda i: (0, 0)),
        ],
        out_specs=pl.BlockSpec((blk, _D), lambda i: (i, 0)),
        out_shape=jax.ShapeDtypeStruct((_N, _D), jnp.float32),
    )(x, w, b.reshape(1, _D))


def _seg_body(hb_hbm, eidx_hbm, zeros_hbm, out_hbm, ibufs, bbufs, fbufs,
              agg, iqs, gsems, ssems):
    c = lax.axis_index("c")
    s = lax.axis_index("s")
    wid = c * _NS + s

    pltpu.sync_copy(zeros_hbm, agg.at[pl.ds(s * _ROWS_PT, _ROWS_PT)])
    plsc.subcore_barrier()

    def ifetch(ci, q):
        pltpu.async_copy(eidx_hbm.at[wid, ci], ibufs[q], iqs[q])

    def wait_ifetch(q):
        pltpu.make_async_copy(eidx_hbm.at[0, 0], ibufs[q], iqs[q]).wait()

    def gather(q, b):
        pltpu.async_copy(hb_hbm.at[ibufs[q].at[0]], bbufs[b], gsems[b])

    def wait_gather(b):
        pltpu.make_async_copy(hb_hbm.at[pl.ds(0, _CHUNK)], bbufs[b],
                              gsems[b]).wait()

    def widen(b):
        bbuf, fbuf = bbufs[b], fbufs[b]

        sh16 = jnp.full((16,), 16, dtype=jnp.int32)
        mask = jnp.full((16,), -65536, dtype=jnp.int32)

        @pl.loop(0, _CHUNK)
        def _row(r):
            for g in range(_D // 32):
                w = bbuf[r, pl.ds(g * 16, 16)]
                lo = lax.bitcast_convert_type(jnp.left_shift(w, sh16), jnp.float32)
                hi = lax.bitcast_convert_type(jnp.bitwise_and(w, mask), jnp.float32)
                fbuf[r, pl.ds(g * 32, 16)] = lo
                fbuf[r, pl.ds(g * 32 + 16, 16)] = hi

    def scatter(q, b):
        pltpu.async_copy(fbufs[b], agg.at[ibufs[q].at[1]], ssems[b],
                         add=True)

    def wait_scatter(b):
        pltpu.make_async_copy(hb_hbm.at[pl.ds(0, _CHUNK)], fbufs[b],
                              ssems[b]).wait()

    for q in range(_NBUF):
        ifetch(q, q)
    wait_ifetch(0)
    gather(0, 0)
    wait_ifetch(1)
    gather(1, 1)
    for n in range(_NIB):
        b = n % _NBUF
        if n >= _NBUF:
            wait_scatter(b)
        wait_gather(b)
        widen(b)
        scatter(n % _NIB, b)
        ifetch(n + _NBUF, (n + _NBUF) % _NIB)
        wait_ifetch((n + 2) % _NIB)
        gather((n + 2) % _NIB, (n + 2) % _NBUF)

    @pl.loop(1, _NCHUNK // _NIB)
    def _steady(t):
        for jj in range(_NIB):
            n = t * _NIB + jj
            b = jj % _NBUF
            wait_scatter(b)
            wait_gather(b)
            widen(b)
            scatter(jj, b)

            @pl.when(n + _NBUF < _NCHUNK)
            def _():
                ifetch(n + _NBUF, (jj + _NBUF) % _NIB)

            @pl.when(n + 2 < _NCHUNK)
            def _():
                wait_ifetch((jj + 2) % _NIB)
                gather((jj + 2) % _NIB, (jj + 2) % _NBUF)

    for b in range(_NBUF):
        wait_scatter(b)

    plsc.subcore_barrier()

    pltpu.sync_copy(
        agg.at[pl.ds(s * _ROWS_PT, _ROWS_PT)],
        out_hbm.at[c, pl.ds(s * _ROWS_PT, _ROWS_PT)],
    )


_seg = functools.partial(
    pl.kernel,
    out_type=jax.ShapeDtypeStruct((_NC, _NPAD, _D), jnp.float32),
    mesh=plsc.VectorSubcoreMesh(core_axis_name="c", subcore_axis_name="s"),
    compiler_params=pltpu.CompilerParams(use_tc_tiling_on_sc=False),
    scratch_types=[
        [pltpu.VMEM((2, _CHUNK), jnp.int32) for _ in range(_NIB)],
        [pltpu.VMEM((_CHUNK, _D // 2), jnp.int32) for _ in range(_NBUF)],
        [pltpu.VMEM((_CHUNK, _D), jnp.float32) for _ in range(_NBUF)],
        pltpu.VMEM_SHARED((_NPAD, _D), jnp.float32),
        [pltpu.SemaphoreType.DMA for _ in range(_NIB)],
        [pltpu.SemaphoreType.DMA for _ in range(_NBUF)],
        [pltpu.SemaphoreType.DMA for _ in range(_NBUF)],
    ],
)(_seg_body)


def _tail_body(p0_ref, p1_ref, h_ref, wrel_ref, brel_ref, wroot_ref,
               wproj_ref, bproj_ref, o_ref):
    agg = p0_ref[...] + p1_ref[...]
    t = jnp.tanh(
        jnp.dot(agg, wrel_ref[...], preferred_element_type=jnp.float32)
        + brel_ref[...]
        + jnp.dot(h_ref[...], wroot_ref[...], preferred_element_type=jnp.float32)
    )
    o_ref[...] = (
        jnp.dot(t, wproj_ref[...], preferred_element_type=jnp.float32)
        + bproj_ref[...]
    )


def _tail(p0, p1, h, w_rel, b_rel, w_root, w_proj, b_proj):
    blk = 1000
    full = pl.BlockSpec((_D, _D), lambda i: (0, 0))
    bias = pl.BlockSpec((1, _D), lambda i: (0, 0))
    row = pl.BlockSpec((blk, _D), lambda i: (i, 0))
    return pl.pallas_call(
        _tail_body,
        grid=(_N // blk,),
        in_specs=[row, row, row, full, bias, full, full, bias],
        out_specs=row,
        out_shape=jax.ShapeDtypeStruct((_N, _D), jnp.float32),
    )(p0, p1, h, w_rel, b_rel.reshape(1, _D), w_root, w_proj,
      b_proj.reshape(1, _D))


def kernel(x, edge_index, W_lift, b_lift, W_rel, b_rel, W_root, W_proj,
           b_proj):
    nw = _NC * _NS
    pad = _EPAD - _EP_TILE
    srcp = jnp.pad(edge_index[0].astype(jnp.int32).reshape(nw, _EP_TILE),
                   ((0, 0), (0, pad)))
    dstp = jnp.pad(edge_index[1].astype(jnp.int32).reshape(nw, _EP_TILE),
                   ((0, 0), (0, pad)), constant_values=_DUMP)
    eidx = jnp.concatenate([
        srcp.reshape(nw, _NCHUNK, 1, _CHUNK),
        dstp.reshape(nw, _NCHUNK, 1, _CHUNK),
    ], axis=2)
    h = _lift(x, W_lift, b_lift)
    hb = jax.lax.bitcast_convert_type(
        h.astype(jnp.bfloat16)
         .reshape(_N, _D // 32, 2, 16)
         .swapaxes(2, 3)
         .reshape(_N, _D // 2, 2),
        jnp.int32)
    zeros = jnp.zeros((_ROWS_PT, _D), jnp.float32)
    partials = _seg(hb, eidx, zeros)
    return _tail(partials[0], partials[1], h, W_rel, b_rel, W_root, W_proj,
                 b_proj)


# speedup vs baseline: 1.6334x; 1.6334x over previous
import functools

import jax
import jax.numpy as jnp
from jax import lax
from jax.experimental import pallas as pl
from jax.experimental.pallas import tpu as pltpu
from jax.experimental.pallas import tpu_sc as plsc

_N = 10000
_E = 320000
_D = 128

_NC = 2
_NS = 16
_EP_TILE = _E // (_NC * _NS)
_CHUNK = 80
_NCHUNK = 126
_EPAD = _NCHUNK * _CHUNK
_NPAD = 10240
_ROWS_PT = _NPAD // _NS
_DUMP = _NPAD - 1

_NBUF = 3
_NIB = 2 * _NBUF


def _seg_body(x_hbm, eidx_hbm, zeros_hbm, zvec_hbm, out_hbm, outd_hbm,
              ibufs, rows, ones, agg, deg, iqs, gsems, ssems):
    c = lax.axis_index("c")
    s = lax.axis_index("s")
    wid = c * _NS + s

    pltpu.sync_copy(zeros_hbm, agg.at[pl.ds(s * _ROWS_PT, _ROWS_PT)])
    pltpu.sync_copy(zvec_hbm, deg.at[pl.ds(s * _ROWS_PT, _ROWS_PT)])
    for k in range(_CHUNK // 16):
        ones[pl.ds(k * 16, 16)] = jnp.ones((16,), jnp.float32)
    plsc.subcore_barrier()

    def ifetch(ci, q):
        pltpu.async_copy(eidx_hbm.at[wid, ci], ibufs[q], iqs[q])

    def wait_ifetch(q):
        pltpu.make_async_copy(eidx_hbm.at[0, 0], ibufs[q], iqs[q]).wait()

    def gather(q, b):
        pltpu.async_copy(x_hbm.at[ibufs[q].at[0]], rows[b], gsems[b])

    def wait_gather(b):
        pltpu.make_async_copy(x_hbm.at[pl.ds(0, _CHUNK)], rows[b],
                              gsems[b]).wait()

    def scatter(q, b):
        pltpu.async_copy(rows[b], agg.at[ibufs[q].at[1]], ssems[b],
                         add=True)
        pltpu.async_copy(ones, deg.at[ibufs[q].at[1]], ssems[b], add=True)

    def wait_scatter(b):
        pltpu.make_async_copy(x_hbm.at[pl.ds(0, _CHUNK)], rows[b],
                              ssems[b]).wait()
        pltpu.make_async_copy(zvec_hbm.at[pl.ds(0, _CHUNK)], ones,
                              ssems[b]).wait()

    for q in range(_NBUF):
        ifetch(q, q)
    for n in range(_NIB):
        b = n % _NBUF
        if n >= _NBUF:
            wait_scatter(b)
        ifetch(n + _NBUF, (n + _NBUF) % _NIB)
        if n >= 1:
            pb = (n - 1) % _NBUF
            wait_gather(pb)
            scatter((n - 1) % _NIB, pb)
        wait_ifetch(n % _NIB)
        gather(n % _NIB, b)

    @pl.loop(1, _NCHUNK // _NIB)
    def _steady(t):
        for jj in range(_NIB):
            n = t * _NIB + jj
            b = jj % _NBUF
            wait_scatter(b)
            nxt = n + _NBUF

            @pl.when(nxt < _NCHUNK)
            def _():
                ifetch(nxt, (jj + _NBUF) % _NIB)

            pb = (jj - 1) % _NBUF
            wait_gather(pb)
            scatter((jj - 1) % _NIB, pb)
            wait_ifetch(jj)
            gather(jj, b)

    lb = (_NCHUNK - 1) % _NBUF
    wait_gather(lb)
    scatter((_NCHUNK - 1) % _NIB, lb)
    for b in range(_NBUF):
        wait_scatter(b)

    plsc.subcore_barrier()

    pltpu.sync_copy(
        agg.at[pl.ds(s * _ROWS_PT, _ROWS_PT)],
        out_hbm.at[c, pl.ds(s * _ROWS_PT, _ROWS_PT)],
    )
    pltpu.sync_copy(
        deg.at[pl.ds(s * _ROWS_PT, _ROWS_PT)],
        outd_hbm.at[c, pl.ds(s * _ROWS_PT, _ROWS_PT)],
    )


_seg = functools.partial(
    pl.kernel,
    out_type=(
        jax.ShapeDtypeStruct((_NC, _NPAD, _D), jnp.float32),
        jax.ShapeDtypeStruct((_NC, _NPAD), jnp.float32),
    ),
    mesh=plsc.VectorSubcoreMesh(core_axis_name="c", subcore_axis_name="s"),
    scratch_types=[
        [pltpu.VMEM((2, _CHUNK), jnp.int32) for _ in range(_NIB)],
        [pltpu.VMEM((_CHUNK, _D), jnp.float32) for _ in range(_NBUF)],
        pltpu.VMEM((_CHUNK,), jnp.float32),
        pltpu.VMEM_SHARED((_NPAD, _D), jnp.float32),
        pltpu.VMEM_SHARED((_NPAD,), jnp.float32),
        [pltpu.SemaphoreType.DMA for _ in range(_NIB)],
        [pltpu.SemaphoreType.DMA for _ in range(_NBUF)],
        [pltpu.SemaphoreType.DMA for _ in range(_NBUF)],
    ],
)(_seg_body)


_BLK = 1280


def _tail_body(p0_ref, p1_ref, d_ref, x_ref, wlift_ref, blift_ref,
               wrel_ref, brel_ref, wroot_ref, wproj_ref, bproj_ref, o_ref):
    i = pl.program_id(0)
    wlift = wlift_ref[...]
    wc_rel = jnp.dot(wlift, wrel_ref[...], preferred_element_type=jnp.float32)
    wc_root = jnp.dot(wlift, wroot_ref[...],
                      preferred_element_type=jnp.float32)
    v1 = jnp.dot(blift_ref[...], wrel_ref[...],
                 preferred_element_type=jnp.float32)
    b2 = brel_ref[...] + jnp.dot(blift_ref[...], wroot_ref[...],
                                 preferred_element_type=jnp.float32)
    aggx = p0_ref[...] + p1_ref[...]
    off = pl.multiple_of(i * _BLK, 128)
    deg = (d_ref[0, pl.ds(off, _BLK)]
           + d_ref[1, pl.ds(off, _BLK)]).reshape(_BLK, 1)
    t = jnp.tanh(
        jnp.dot(aggx, wc_rel, preferred_element_type=jnp.float32)
        + deg * v1
        + jnp.dot(x_ref[...], wc_root, preferred_element_type=jnp.float32)
        + b2
    )
    o_ref[...] = (
        jnp.dot(t, wproj_ref[...], preferred_element_type=jnp.float32)
        + bproj_ref[...]
    )


def _tail(p0, p1, d, x, w_lift, b_lift, w_rel, b_rel, w_root, w_proj,
          b_proj):
    full = pl.BlockSpec((_D, _D), lambda i: (0, 0))
    bias = pl.BlockSpec((1, _D), lambda i: (0, 0))
    row = pl.BlockSpec((_BLK, _D), lambda i: (i, 0))
    dspec = pl.BlockSpec((_NC, _NPAD), lambda i: (0, 0))
    return pl.pallas_call(
        _tail_body,
        grid=(_NPAD // _BLK,),
        in_specs=[row, row, dspec, row, full, bias, full, bias, full,
                  full, bias],
        out_specs=row,
        out_shape=jax.ShapeDtypeStruct((_NPAD, _D), jnp.float32),
    )(p0, p1, d, x, w_lift, b_lift.reshape(1, _D), w_rel,
      b_rel.reshape(1, _D), w_root, w_proj, b_proj.reshape(1, _D))


def kernel(x, edge_index, W_lift, b_lift, W_rel, b_rel, W_root, W_proj,
           b_proj):
    nw = _NC * _NS
    pad = _EPAD - _EP_TILE
    srcp = jnp.pad(edge_index[0].astype(jnp.int32).reshape(nw, _EP_TILE),
                   ((0, 0), (0, pad)))
    dstp = jnp.pad(edge_index[1].astype(jnp.int32).reshape(nw, _EP_TILE),
                   ((0, 0), (0, pad)), constant_values=_DUMP)
    eidx = jnp.concatenate([
        srcp.reshape(nw, _NCHUNK, 1, _CHUNK),
        dstp.reshape(nw, _NCHUNK, 1, _CHUNK),
    ], axis=2)
    zeros = jnp.zeros((_ROWS_PT, _D), jnp.float32)
    zvec = jnp.zeros((_ROWS_PT,), jnp.float32)
    partials, degp = _seg(x, eidx, zeros, zvec)
    xp = jnp.pad(x, ((0, _NPAD - _N), (0, 0)))
    out = _tail(partials[0], partials[1], degp, xp, W_lift, b_lift, W_rel,
                b_rel, W_root, W_proj, b_proj)
    return out[:_N]


# speedup vs baseline: 1.6416x; 1.0050x over previous
import functools

import jax
import jax.numpy as jnp
from jax import lax
from jax.experimental import pallas as pl
from jax.experimental.pallas import tpu as pltpu
from jax.experimental.pallas import tpu_sc as plsc

_N = 10000
_E = 320000
_D = 128

_NC = 2
_NS = 16
_EP_CORE = _E // _NC
_EP_TILE = _EP_CORE // _NS
_CHUNK = 80
_NCHUNK = 126
_NPAD = 10240
_ROWS_PT = _NPAD // _NS
_DUMP = _NPAD - 1

_NBUF = 3
_NIB = 2 * _NBUF


def _lift_body(x_ref, w_ref, b_ref, o_ref):
    o_ref[...] = (
        jnp.dot(x_ref[...], w_ref[...], preferred_element_type=jnp.float32)
        + b_ref[...]
    )


def _lift(x, w, b):
    blk = 1000
    return pl.pallas_call(
        _lift_body,
        grid=(_N // blk,),
        in_specs=[
            pl.BlockSpec((blk, _D), lambda i: (i, 0)),
            pl.BlockSpec((_D, _D), lambda i: (0, 0)),
            pl.BlockSpec((1, _D), lambda i: (0, 0)),
        ],
        out_specs=pl.BlockSpec((blk, _D), lambda i: (i, 0)),
        out_shape=jax.ShapeDtypeStruct((_N, _D), jnp.float32),
    )(x, w, b.reshape(1, _D))


def _seg_body(h_hbm, eidx_hbm, zeros_hbm, out_hbm, ibufs, rows, agg,
              iqs, gsems, ssems):
    c = lax.axis_index("c")
    s = lax.axis_index("s")
    wid = c * _NS + s

    pltpu.sync_copy(zeros_hbm, agg.at[pl.ds(s * _ROWS_PT, _ROWS_PT)])
    plsc.subcore_barrier()

    def ifetch(ci, q):
        pltpu.async_copy(eidx_hbm.at[wid, ci], ibufs[q], iqs[q])

    def wait_ifetch(q):
        pltpu.make_async_copy(eidx_hbm.at[0, 0], ibufs[q], iqs[q]).wait()

    def gather(q, b):
        pltpu.async_copy(h_hbm.at[ibufs[q].at[0]], rows[b], gsems[b])

    def wait_gather(b):
        pltpu.make_async_copy(h_hbm.at[pl.ds(0, _CHUNK)], rows[b],
                              gsems[b]).wait()

    def scatter(q, b):
        pltpu.async_copy(rows[b], agg.at[ibufs[q].at[1]], ssems[b],
                         add=True)

    def wait_scatter(b):
        pltpu.make_async_copy(h_hbm.at[pl.ds(0, _CHUNK)], rows[b],
                              ssems[b]).wait()

    for q in range(_NBUF):
        ifetch(q, q)
    for n in range(_NIB):
        b = n % _NBUF
        if n >= _NBUF:
            wait_scatter(b)
        ifetch(n + _NBUF, (n + _NBUF) % _NIB)
        if n >= 1:
            pb = (n - 1) % _NBUF
            wait_gather(pb)
            scatter((n - 1) % _NIB, pb)
        wait_ifetch(n % _NIB)
        gather(n % _NIB, b)

    @pl.loop(1, _NCHUNK // _NIB)
    def _steady(t):
        for jj in range(_NIB):
            n = t * _NIB + jj
            b = jj % _NBUF
            wait_scatter(b)
            nxt = n + _NBUF

            @pl.when(nxt < _NCHUNK)
            def _():
                ifetch(nxt, (jj + _NBUF) % _NIB)

            pb = (jj - 1) % _NBUF
            wait_gather(pb)
            scatter((jj - 1) % _NIB, pb)
            wait_ifetch(jj)
            gather(jj, b)

    lb = (_NCHUNK - 1) % _NBUF
    wait_gather(lb)
    scatter((_NCHUNK - 1) % _NIB, lb)
    for b in range(_NBUF):
        wait_scatter(b)

    plsc.subcore_barrier()

    pltpu.sync_copy(
        agg.at[pl.ds(s * _ROWS_PT, _ROWS_PT)],
        out_hbm.at[c, pl.ds(s * _ROWS_PT, _ROWS_PT)],
    )


_seg = functools.partial(
    pl.kernel,
    out_type=jax.ShapeDtypeStruct((_NC, _NPAD, _D), jnp.float32),
    mesh=plsc.VectorSubcoreMesh(core_axis_name="c", subcore_axis_name="s"),
    scratch_types=[
        [pltpu.VMEM((2, _CHUNK), jnp.int32) for _ in range(_NIB)],
        [pltpu.VMEM((_CHUNK, _D), jnp.float32) for _ in range(_NBUF)],
        pltpu.VMEM_SHARED((_NPAD, _D), jnp.float32),
        [pltpu.SemaphoreType.DMA for _ in range(_NIB)],
        [pltpu.SemaphoreType.DMA for _ in range(_NBUF)],
        [pltpu.SemaphoreType.DMA for _ in range(_NBUF)],
    ],
)(_seg_body)


def _tail_body(p0_ref, p1_ref, h_ref, wrel_ref, brel_ref, wroot_ref,
               wproj_ref, bproj_ref, o_ref):
    agg = p0_ref[...] + p1_ref[...]
    t = jnp.tanh(
        jnp.dot(agg, wrel_ref[...], preferred_element_type=jnp.float32)
        + brel_ref[...]
        + jnp.dot(h_ref[...], wroot_ref[...], preferred_element_type=jnp.float32)
    )
    o_ref[...] = (
        jnp.dot(t, wproj_ref[...], preferred_element_type=jnp.float32)
        + bproj_ref[...]
    )


def _tail(p0, p1, h, w_rel, b_rel, w_root, w_proj, b_proj):
    blk = 1000
    full = pl.BlockSpec((_D, _D), lambda i: (0, 0))
    bias = pl.BlockSpec((1, _D), lambda i: (0, 0))
    row = pl.BlockSpec((blk, _D), lambda i: (i, 0))
    return pl.pallas_call(
        _tail_body,
        grid=(_N // blk,),
        in_specs=[row, row, row, full, bias, full, full, bias],
        out_specs=row,
        out_shape=jax.ShapeDtypeStruct((_N, _D), jnp.float32),
    )(p0, p1, h, w_rel, b_rel.reshape(1, _D), w_root, w_proj,
      b_proj.reshape(1, _D))


def kernel(x, edge_index, W_lift, b_lift, W_rel, b_rel, W_root, W_proj,
           b_proj):
    nw = _NC * _NS
    src = edge_index[0].astype(jnp.int32).reshape(nw, _NCHUNK - 1, 1, _CHUNK)
    dst = edge_index[1].astype(jnp.int32).reshape(nw, _NCHUNK - 1, 1, _CHUNK)
    pad_src = jnp.zeros((nw, 1, 1, _CHUNK), jnp.int32)
    pad_dst = jnp.full((nw, 1, 1, _CHUNK), _DUMP, jnp.int32)
    eidx = jnp.concatenate([
        jnp.concatenate([src, pad_src], axis=1),
        jnp.concatenate([dst, pad_dst], axis=1),
    ], axis=2)
    h = _lift(x, W_lift, b_lift)
    zeros = jnp.zeros((_ROWS_PT, _D), jnp.float32)
    partials = _seg(h, eidx, zeros)
    return _tail(partials[0], partials[1], h, W_rel, b_rel, W_root, W_proj,
                 b_proj)
